# Initial kernel scaffold; baseline (speedup 1.0000x reference)
import jax
import jax.numpy as jnp
from jax.experimental import pallas as pl


def kernel(ref_mel):
    raise NotImplementedError("write your pallas kernel here")


# speedup vs baseline: 1.9592x; 1.9592x over previous
import jax
import jax.numpy as jnp
from jax.experimental import pallas as pl

B, T, D = 32, 4096, 80
BINS = 24
PADN = 4112


def _energy_kernel(x_ref, o_ref):
    o_ref[...] = jnp.sum(x_ref[...], axis=-1) / jnp.float32(D)


def _shift_right(x, k):
    z = jnp.zeros((x.shape[0], k), x.dtype)
    return jnp.concatenate([z, x[:, :-k]], axis=1)


def _inblock_scan16(x):
    lane = jax.lax.broadcasted_iota(jnp.int32, x.shape, 1) & 15
    acc = x
    for j in range(1, 16):
        acc = acc + jnp.where(lane == j, _shift_right(acc, 1), jnp.float32(0.0))
    return acc


def _dot(a, b):
    return jnp.dot(a, b, precision=jax.lax.Precision.HIGHEST,
                   preferred_element_type=jnp.float32)


def _emulated_cumsum_4112(cin, sel_mats):
    s1, e1m, s2, e2m = sel_mats
    L1 = _inblock_scan16(cin)
    ends1 = _dot(L1, s1)
    e1p = jnp.concatenate(
        [ends1, jnp.zeros((B, 272 - 257), jnp.float32)], axis=1)
    L2 = _inblock_scan16(e1p)
    ends2 = _dot(L2, s2)
    e2p = jnp.concatenate(
        [ends2, jnp.zeros((B, 32 - 17), jnp.float32)], axis=1)
    L3 = _inblock_scan16(e2p)
    off3 = L3[:, 15:16]
    lane32 = jax.lax.broadcasted_iota(jnp.int32, (B, 32), 1)
    off3_full = jnp.where(lane32 < 16, jnp.float32(0.0),
                          jnp.broadcast_to(off3, (B, 32)))
    F3 = L3 + off3_full
    off2 = jnp.concatenate(
        [jnp.zeros((B, 1), jnp.float32), F3[:, :16]], axis=1)
    F2 = L2 + _dot(off2, e2m)
    off1 = jnp.concatenate(
        [jnp.zeros((B, 1), jnp.float32), F2[:, :256]], axis=1)
    F1 = L1 + _dot(off1, e1m)
    return F1


def _order_stat(bits, k):
    def body(_, lohi):
        lo, hi = lohi
        mid = lo + (hi - lo) // 2
        cnt = jnp.sum((bits <= mid).astype(jnp.int32), axis=1, keepdims=True)
        take = cnt >= (k + 1)
        return jnp.where(take, lo, mid + 1), jnp.where(take, mid, hi)
    lo = jnp.zeros((B, 1), jnp.int32)
    hi = jnp.full((B, 1), jnp.int32(2**31 - 1))
    lo, hi = jax.lax.fori_loop(0, 31, body, (lo, hi))
    return jax.lax.bitcast_convert_type(lo, jnp.float32)


def _quantile_thr(bits, q):
    qv = jnp.float32(q) * jnp.float32(T - 1)
    low = jnp.floor(qv)
    w_hi = qv - low
    w_lo = jnp.float32(1.0) - w_hi
    k = int(q * (T - 1))
    s_lo = _order_stat(bits, k)
    s_hi = _order_stat(bits, k + 1)
    return s_lo * w_lo + s_hi * w_hi


def _main_kernel(energy_ref, uniform_ref, tp_ref,
                 f0_o, f1_o, f2_o, f3_o, f4_o, stats_o):
    f32 = jnp.float32
    energy = energy_ref[...]
    uniform = uniform_ref[...]
    tp = tp_ref[...]

    em = jnp.sum(energy, axis=1, keepdims=True) / f32(T)
    cen = energy - em
    var = jnp.sum(cen * cen, axis=1, keepdims=True) / f32(T - 1)
    es = jnp.maximum(jnp.sqrt(var), f32(1e-6))
    ez = (energy - em) / es

    dif = jnp.abs(energy[:, 1:] - energy[:, :-1])
    delta = jnp.concatenate([jnp.zeros((B, 1), f32), dif], axis=1)

    dbits = jax.lax.bitcast_convert_type(delta, jnp.int32)
    dthr = _quantile_thr(dbits, 0.35)

    pause = (ez <= f32(-0.5)) & (delta <= dthr)
    voiced = (ez > f32(-0.1)).astype(f32)

    it_s1 = jax.lax.broadcasted_iota(jnp.int32, (PADN, 257), 0)
    ib_s1 = jax.lax.broadcasted_iota(jnp.int32, (PADN, 257), 1)
    s1 = (it_s1 == 16 * ib_s1 + 15).astype(f32)
    ib_e1 = jax.lax.broadcasted_iota(jnp.int32, (257, PADN), 0)
    it_e1 = jax.lax.broadcasted_iota(jnp.int32, (257, PADN), 1)
    e1m = ((it_e1 >> 4) == ib_e1).astype(f32)
    it_s2 = jax.lax.broadcasted_iota(jnp.int32, (272, 17), 0)
    ib_s2 = jax.lax.broadcasted_iota(jnp.int32, (272, 17), 1)
    s2 = (it_s2 == 16 * ib_s2 + 15).astype(f32)
    ib_e2 = jax.lax.broadcasted_iota(jnp.int32, (17, 272), 0)
    it_e2 = jax.lax.broadcasted_iota(jnp.int32, (17, 272), 1)
    e2m = ((it_e2 >> 4) == ib_e2).astype(f32)
    sel = (s1, e1m, s2, e2m)

    def pool_with(k):
        p = k // 2
        n_real = T + 2 * p + 1
        cin = jnp.concatenate(
            [jnp.zeros((B, p + 1), f32), delta,
             jnp.zeros((B, PADN - T - p - 1), f32)], axis=1)
        c = _emulated_cumsum_4112(cin, sel)
        return (c[:, k:n_real] - c[:, :n_real - k]) / f32(k)

    local_rate = pool_with(5)
    bs = pool_with(7)

    bbits = jax.lax.bitcast_convert_type(bs, jnp.int32)
    bthr = _quantile_thr(bbits, 0.75)
    bev = (bs >= bthr).astype(f32)

    sp = jnp.where(pause, f32(0.0), f32(1.0))
    k = 1
    while k < T:
        sp = sp + _shift_right(sp, k)
        k *= 2
    total = jnp.maximum(sp[:, T - 1:T], f32(1.0))
    progress = sp / total
    sdb = progress - uniform

    pause_f = pause.astype(f32)
    feats = (pause_f, local_rate, bev, sdb, voiced)

    iota_t = jax.lax.broadcasted_iota(jnp.int32, (B, T), 1)
    outs = [[] for _ in range(5)]
    for j in range(BINS):
        tpj = tp[:, j:j + 1]
        right = jnp.sum((progress < tpj).astype(jnp.int32),
                        axis=1, keepdims=True)
        left = jnp.clip(right - 1, 0, T - 1)
        r = jnp.clip(right, 0, T - 1)
        oh_l = (iota_t == left).astype(f32)
        oh_r = (iota_t == r).astype(f32)
        lp = jnp.sum(oh_l * progress, axis=1, keepdims=True)
        rp = jnp.sum(oh_r * progress, axis=1, keepdims=True)
        denom = jnp.maximum(jnp.abs(rp - lp), f32(1e-6))
        alpha = jnp.clip((tpj - lp) / denom, f32(0.0), f32(1.0))
        lo_edge = right <= 0
        hi_edge = right >= T
        for d, fd in enumerate(feats):
            v_l = jnp.sum(oh_l * fd, axis=1, keepdims=True)
            v_r = jnp.sum(oh_r * fd, axis=1, keepdims=True)
            val = v_l * (f32(1.0) - alpha) + v_r * alpha
            val = jnp.where(lo_edge, fd[:, 0:1], val)
            val = jnp.where(hi_edge, fd[:, T - 1:T], val)
            outs[d].append(val)
    for d, o_ref in enumerate((f0_o, f1_o, f2_o, f3_o, f4_o)):
        o_ref[...] = jnp.concatenate(outs[d], axis=1)

    half = T // 2
    rate_trend = (jnp.sum(local_rate[:, half:], axis=1, keepdims=True) / f32(half)
                  - jnp.sum(local_rate[:, :half], axis=1, keepdims=True) / f32(half))

    def run_mean(mask_i):
        prev = jnp.concatenate(
            [jnp.zeros((B, 1), jnp.int32), mask_i[:, :-1]], axis=1)
        starts = jnp.sum(((mask_i == 1) & (prev == 0)).astype(jnp.int32),
                         axis=1, keepdims=True)
        tot = jnp.sum(mask_i, axis=1, keepdims=True)
        return tot.astype(f32) / jnp.maximum(starts, 1).astype(f32)

    pause_i = pause.astype(jnp.int32)
    speech_i = 1 - pause_i
    stats_o[...] = jnp.concatenate([
        jnp.sum(pause_f, axis=1, keepdims=True) / f32(T),
        run_mean(pause_i),
        run_mean(speech_i),
        rate_trend,
        jnp.sum(bev, axis=1, keepdims=True) / f32(T),
        jnp.sum(voiced, axis=1, keepdims=True) / f32(T),
    ], axis=1)


def kernel(ref_mel):
    ref_mel = ref_mel.astype(jnp.float32)
    energy = pl.pallas_call(
        _energy_kernel,
        grid=(4,),
        in_specs=[pl.BlockSpec((8, T, D), lambda i: (i, 0, 0))],
        out_specs=pl.BlockSpec((8, T), lambda i: (i, 0)),
        out_shape=jax.ShapeDtypeStruct((B, T), jnp.float32),
    )(ref_mel)

    uniform = jnp.linspace(0.0, 1.0, T)[None, :]
    tp = jnp.linspace(0.0, 1.0, BINS)[None, :]

    shapes = [jax.ShapeDtypeStruct((B, BINS), jnp.float32) for _ in range(5)]
    shapes.append(jax.ShapeDtypeStruct((B, 6), jnp.float32))
    f0, f1, f2, f3, f4, stats = pl.pallas_call(
        _main_kernel,
        out_shape=tuple(shapes),
    )(energy, uniform, tp)

    trace = jnp.stack([f0, f1, f2, f3, f4], axis=-1)
    return trace, stats
